# baseline (speedup 1.0000x reference)
import jax
import jax.numpy as jnp
from jax.experimental import pallas as pl
from jax.experimental.pallas import tpu as pltpu

_B = 16
_H = 16
_S = 2048
_D = 64


def _body(pos_ref, kc_ref, vc_ref, kval_ref, vval_ref, kout_ref, vout_ref,
          copy_sem, row_sem):
    big_k = pltpu.make_async_copy(kc_ref, kout_ref, copy_sem)
    big_v = pltpu.make_async_copy(vc_ref, vout_ref, copy_sem)
    big_k.start()
    big_v.start()
    big_k.wait()
    big_v.wait()
    for b in range(_B):
        r = pos_ref[b] - 1
        pltpu.make_async_copy(
            kval_ref.at[b], kout_ref.at[b, :, pl.ds(r, 1), :], row_sem
        ).start()
        pltpu.make_async_copy(
            vval_ref.at[b], vout_ref.at[b, :, pl.ds(r, 1), :], row_sem
        ).start()
    for b in range(_B):
        r = pos_ref[b] - 1
        pltpu.make_async_copy(
            kval_ref.at[b], kout_ref.at[b, :, pl.ds(r, 1), :], row_sem
        ).wait()
        pltpu.make_async_copy(
            vval_ref.at[b], vout_ref.at[b, :, pl.ds(r, 1), :], row_sem
        ).wait()


def kernel(k_cache, v_cache, k_val, v_val, input_pos):
    out_shape = jax.ShapeDtypeStruct((_B, _H, _S, _D), jnp.float32)
    any_spec = pl.BlockSpec(memory_space=pltpu.MemorySpace.HBM)
    grid_spec = pltpu.PrefetchScalarGridSpec(
        num_scalar_prefetch=1,
        grid=(),
        in_specs=[
            any_spec,
            any_spec,
            pl.BlockSpec(memory_space=pltpu.MemorySpace.VMEM),
            pl.BlockSpec(memory_space=pltpu.MemorySpace.VMEM),
        ],
        out_specs=[any_spec, any_spec],
        scratch_shapes=[pltpu.SemaphoreType.DMA, pltpu.SemaphoreType.DMA],
    )
    return pl.pallas_call(
        _body,
        grid_spec=grid_spec,
        out_shape=[out_shape, out_shape],
    )(input_pos, k_cache, v_cache, k_val, v_val)


# speedup vs baseline: 23.4401x; 23.4401x over previous
import jax
import jax.numpy as jnp
from jax.experimental import pallas as pl
from jax.experimental.pallas import tpu as pltpu

_B = 16
_H = 16
_S = 2048
_D = 64


def _body(pos_ref, kc_ref, vc_ref, kval_ref, vval_ref, kout_ref, vout_ref):
    b = pl.program_id(0)
    r = (pos_ref[b] - 1) % 8
    kout_ref[...] = kc_ref[...]
    vout_ref[...] = vc_ref[...]
    kout_ref[:, pl.ds(r, 1), :] = kval_ref[...]
    vout_ref[:, pl.ds(r, 1), :] = vval_ref[...]


def kernel(k_cache, v_cache, k_val, v_val, input_pos):
    out_shape = jax.ShapeDtypeStruct((_B, _H, _S, _D), jnp.float32)
    slab_spec = pl.BlockSpec(
        (None, _H, 8, _D), lambda b, pos: (b, 0, (pos[b] - 1) // 8, 0)
    )
    val_spec = pl.BlockSpec((None, _H, 1, _D), lambda b, pos: (b, 0, 0, 0))
    grid_spec = pltpu.PrefetchScalarGridSpec(
        num_scalar_prefetch=1,
        grid=(_B,),
        in_specs=[slab_spec, slab_spec, val_spec, val_spec],
        out_specs=[slab_spec, slab_spec],
    )
    return pl.pallas_call(
        _body,
        grid_spec=grid_spec,
        out_shape=[out_shape, out_shape],
        input_output_aliases={1: 0, 2: 1},
    )(input_pos, k_cache, v_cache, k_val, v_val)
